# baseline (speedup 1.0000x reference)
import functools

import jax
import jax.numpy as jnp
from jax.experimental import pallas as pl
from jax.experimental.pallas import tpu as pltpu

S = 4096
E = 64
D = 4096
CAP = 64


def _gate_kernel(x_ref, w_ref, combine_ref, dispatch_ref, idx_ref, loc_ref,
                 gate_ref, laux_ref, counts_ref, me_ref, *, r, nsteps):
    i = pl.program_id(0)

    @pl.when(i == 0)
    def _init():
        counts_ref[...] = jnp.zeros_like(counts_ref)
        me_ref[...] = jnp.zeros_like(me_ref)

    x = x_ref[...]
    w = w_ref[...]
    logits = jax.lax.dot_general(
        x, w, (((1,), (1,)), ((), ())), preferred_element_type=jnp.float32)
    rm = jnp.max(logits, axis=1, keepdims=True)
    unn = jnp.exp(logits - rm)
    den = jnp.sum(unn, axis=1, keepdims=True)
    gates = unn / den

    gmax = jnp.max(gates, axis=1, keepdims=True)
    cols = jax.lax.broadcasted_iota(jnp.int32, (r, E), 1)
    idx = jnp.min(jnp.where(gates == gmax, cols, E), axis=1, keepdims=True)

    maskf = (cols == idx).astype(jnp.float32)
    ri = jax.lax.broadcasted_iota(jnp.int32, (r, r), 0)
    ci = jax.lax.broadcasted_iota(jnp.int32, (r, r), 1)
    tri = (ri >= ci).astype(jnp.float32)
    csum = jax.lax.dot_general(
        tri, maskf, (((1,), (0,)), ((), ())), preferred_element_type=jnp.float32)

    counts = counts_ref[...]
    loc_all = csum - 1.0 + counts
    loc_tok = jnp.sum(loc_all * maskf, axis=1, keepdims=True)

    counts_ref[...] = counts + jnp.sum(maskf, axis=0, keepdims=True)
    me_ref[...] = me_ref[...] + jnp.sum(gates, axis=0, keepdims=True)

    keep = loc_tok < float(CAP)
    g1 = jnp.where(keep, gmax, 0.0)
    loc_i = loc_tok.astype(jnp.int32)
    loc_kept = jnp.where(keep, loc_i, 0)

    gef = jnp.where((cols == idx) & keep, g1, 0.0)
    eqcf = (cols == loc_kept).astype(jnp.float32)
    comb = gef[:, :, None] * eqcf[:, None, :]
    combine_ref[...] = comb
    dispatch_ref[...] = comb > 0.0

    idx_ref[...] = idx
    loc_ref[...] = loc_i
    gate_ref[...] = gmax

    @pl.when(i == nsteps - 1)
    def _fin():
        me = me_ref[...]
        cnt = counts_ref[...]
        laux_ref[...] = (jnp.sum(me * cnt, axis=1, keepdims=True)
                         * (float(E) / (float(S) * float(S))))


@functools.partial(jax.jit, static_argnames=())
def kernel(input, W):
    r = 256
    nsteps = S // r
    grid = (nsteps,)
    out = pl.pallas_call(
        functools.partial(_gate_kernel, r=r, nsteps=nsteps),
        grid=grid,
        in_specs=[
            pl.BlockSpec((r, D), lambda i: (i, 0)),
            pl.BlockSpec((E, D), lambda i: (0, 0)),
        ],
        out_specs=[
            pl.BlockSpec((r, E, CAP), lambda i: (i, 0, 0)),
            pl.BlockSpec((r, E, CAP), lambda i: (i, 0, 0)),
            pl.BlockSpec((r, 1), lambda i: (i, 0)),
            pl.BlockSpec((r, 1), lambda i: (i, 0)),
            pl.BlockSpec((r, 1), lambda i: (i, 0)),
            pl.BlockSpec((1, 1), lambda i: (0, 0)),
        ],
        out_shape=[
            jax.ShapeDtypeStruct((S, E, CAP), jnp.float32),
            jax.ShapeDtypeStruct((S, E, CAP), jnp.bool_),
            jax.ShapeDtypeStruct((S, 1), jnp.int32),
            jax.ShapeDtypeStruct((S, 1), jnp.int32),
            jax.ShapeDtypeStruct((S, 1), jnp.float32),
            jax.ShapeDtypeStruct((1, 1), jnp.float32),
        ],
        scratch_shapes=[
            pltpu.VMEM((1, E), jnp.float32),
            pltpu.VMEM((1, E), jnp.float32),
        ],
    )(input, W)
    combine, dispatch, idx, loc, g1s, laux = out
    return (laux[0, 0], combine, dispatch, idx.reshape(S), loc.reshape(S),
            g1s.reshape(S))


# speedup vs baseline: 3.8588x; 3.8588x over previous
import functools

import jax
import jax.numpy as jnp
from jax.experimental import pallas as pl
from jax.experimental.pallas import tpu as pltpu

S = 4096
E = 64
D = 4096
CAP = 64


def _gate_kernel(x_ref, w_ref, combine_ref, dispatch_ref, idx_ref, loc_ref,
                 gate_ref, laux_ref, counts_ref, me_ref, *, r, nsteps):
    i = pl.program_id(0)

    @pl.when(i == 0)
    def _init():
        counts_ref[...] = jnp.zeros_like(counts_ref)
        me_ref[...] = jnp.zeros_like(me_ref)

    x = x_ref[...]
    w = w_ref[...]
    logits = jax.lax.dot_general(
        w, x, (((1,), (1,)), ((), ())), preferred_element_type=jnp.float32)
    rm = jnp.max(logits, axis=0, keepdims=True)
    unn = jnp.exp(logits - rm)
    den = jnp.sum(unn, axis=0, keepdims=True)
    gates = unn / den

    gmax = jnp.max(gates, axis=0, keepdims=True)
    rows = jax.lax.broadcasted_iota(jnp.int32, (E, r), 0)
    idx = jnp.min(jnp.where(gates == gmax, rows, E), axis=0, keepdims=True)

    maskf = (rows == idx).astype(jnp.float32)
    ri = jax.lax.broadcasted_iota(jnp.int32, (r, r), 0)
    ci = jax.lax.broadcasted_iota(jnp.int32, (r, r), 1)
    tri = (ri <= ci).astype(jnp.float32)
    csum = jax.lax.dot_general(
        maskf, tri, (((1,), (0,)), ((), ())), preferred_element_type=jnp.float32)

    counts = counts_ref[...]
    loc_all = csum - 1.0 + counts
    loc_tok = jnp.sum(loc_all * maskf, axis=0, keepdims=True)

    counts_ref[...] = counts + jnp.sum(maskf, axis=1, keepdims=True)
    me_ref[...] = me_ref[...] + jnp.sum(gates, axis=1, keepdims=True)

    keep = loc_tok < float(CAP)
    loc_i = loc_tok.astype(jnp.int32)
    loc_kept = jnp.where(keep, loc_i, -1)
    idx_k = jnp.where(keep, idx, -1)
    g1 = jnp.where(keep, gmax, 0.0)

    e3 = jax.lax.broadcasted_iota(jnp.int32, (E, CAP, r), 0)
    c3 = jax.lax.broadcasted_iota(jnp.int32, (E, CAP, r), 1)
    m3 = (e3 == idx_k[:, None, :]) & (c3 == loc_kept[:, None, :])
    combine_ref[...] = jnp.where(m3, g1[:, None, :], 0.0)
    dispatch_ref[...] = m3.astype(jnp.int8)

    idx_ref[...] = idx
    loc_ref[...] = loc_i
    gate_ref[...] = gmax

    @pl.when(i == nsteps - 1)
    def _fin():
        me = me_ref[...]
        cnt = counts_ref[...]
        laux_ref[...] = (jnp.sum(me * cnt, axis=0, keepdims=True)
                         * (float(E) / (float(S) * float(S))))


def kernel(input, W):
    r = 256
    nsteps = S // r
    grid = (nsteps,)
    out = pl.pallas_call(
        functools.partial(_gate_kernel, r=r, nsteps=nsteps),
        grid=grid,
        in_specs=[
            pl.BlockSpec((r, D), lambda i: (i, 0)),
            pl.BlockSpec((E, D), lambda i: (0, 0)),
        ],
        out_specs=[
            pl.BlockSpec((E, CAP, r), lambda i: (0, 0, i)),
            pl.BlockSpec((E, CAP, r), lambda i: (0, 0, i)),
            pl.BlockSpec((1, r), lambda i: (0, i)),
            pl.BlockSpec((1, r), lambda i: (0, i)),
            pl.BlockSpec((1, r), lambda i: (0, i)),
            pl.BlockSpec((1, 1), lambda i: (0, 0)),
        ],
        out_shape=[
            jax.ShapeDtypeStruct((E, CAP, S), jnp.float32),
            jax.ShapeDtypeStruct((E, CAP, S), jnp.int8),
            jax.ShapeDtypeStruct((1, S), jnp.int32),
            jax.ShapeDtypeStruct((1, S), jnp.int32),
            jax.ShapeDtypeStruct((1, S), jnp.float32),
            jax.ShapeDtypeStruct((1, 1), jnp.float32),
        ],
        scratch_shapes=[
            pltpu.VMEM((E, 1), jnp.float32),
            pltpu.VMEM((E, 1), jnp.float32),
        ],
    )(input, W)
    combine_t, dispatch_t, idx, loc, g1s, laux = out
    combine = jnp.transpose(combine_t, (2, 0, 1))
    dispatch = jnp.transpose(dispatch_t != 0, (2, 0, 1))
    return (laux[0, 0], combine, dispatch, idx.reshape(S), loc.reshape(S),
            g1s.reshape(S))


# speedup vs baseline: 3.9915x; 1.0344x over previous
import functools

import jax
import jax.numpy as jnp
from jax.experimental import pallas as pl
from jax.experimental.pallas import tpu as pltpu

S = 4096
E = 64
D = 4096
CAP = 64


def _gate_kernel(x_ref, w_ref, combine_ref, dispatch_ref, idx_ref, loc_ref,
                 gate_ref, laux_ref, counts_ref, me_ref, *, r, nsteps):
    i = pl.program_id(0)

    @pl.when(i == 0)
    def _init():
        counts_ref[...] = jnp.zeros_like(counts_ref)
        me_ref[...] = jnp.zeros_like(me_ref)

    x = x_ref[...]
    w = w_ref[...]
    logits = jax.lax.dot_general(
        w, x, (((1,), (1,)), ((), ())), preferred_element_type=jnp.float32)
    rm = jnp.max(logits, axis=0, keepdims=True)
    unn = jnp.exp(logits - rm)
    den = jnp.sum(unn, axis=0, keepdims=True)
    gates = unn / den

    gmax = jnp.max(gates, axis=0, keepdims=True)
    rows = jax.lax.broadcasted_iota(jnp.int32, (E, r), 0)
    idx = jnp.min(jnp.where(gates == gmax, rows, E), axis=0, keepdims=True)

    maskf = (rows == idx).astype(jnp.float32)
    ri = jax.lax.broadcasted_iota(jnp.int32, (r, r), 0)
    ci = jax.lax.broadcasted_iota(jnp.int32, (r, r), 1)
    tri = (ri <= ci).astype(jnp.float32)
    csum = jax.lax.dot_general(
        maskf, tri, (((1,), (0,)), ((), ())), preferred_element_type=jnp.float32)

    counts = counts_ref[...]
    loc_all = csum - 1.0 + counts
    loc_tok = jnp.sum(loc_all * maskf, axis=0, keepdims=True)

    counts_ref[...] = counts + jnp.sum(maskf, axis=1, keepdims=True)
    me_ref[...] = me_ref[...] + jnp.sum(gates, axis=1, keepdims=True)

    keep = loc_tok < float(CAP)
    loc_i = loc_tok.astype(jnp.int32)
    loc_kept = jnp.where(keep, loc_i, -1)
    idx_k = jnp.where(keep, idx, -1)
    g1 = jnp.where(keep, gmax, 0.0)

    e3 = jax.lax.broadcasted_iota(jnp.int32, (E, CAP, r), 0)
    c3 = jax.lax.broadcasted_iota(jnp.int32, (E, CAP, r), 1)
    m3 = (e3 == idx_k[:, None, :]) & (c3 == loc_kept[:, None, :])
    combine_ref[...] = jnp.where(m3, g1[:, None, :], 0.0)
    dispatch_ref[...] = m3.astype(jnp.int8)

    idx_ref[...] = idx
    loc_ref[...] = loc_i
    gate_ref[...] = gmax

    @pl.when(i == nsteps - 1)
    def _fin():
        me = me_ref[...]
        cnt = counts_ref[...]
        laux_ref[...] = (jnp.sum(me * cnt, axis=0, keepdims=True)
                         * (float(E) / (float(S) * float(S))))


def kernel(input, W):
    r = 512
    nsteps = S // r
    grid = (nsteps,)
    out = pl.pallas_call(
        functools.partial(_gate_kernel, r=r, nsteps=nsteps),
        grid=grid,
        in_specs=[
            pl.BlockSpec((r, D), lambda i: (i, 0)),
            pl.BlockSpec((E, D), lambda i: (0, 0)),
        ],
        out_specs=[
            pl.BlockSpec((E, CAP, r), lambda i: (0, 0, i)),
            pl.BlockSpec((E, CAP, r), lambda i: (0, 0, i)),
            pl.BlockSpec((1, r), lambda i: (0, i)),
            pl.BlockSpec((1, r), lambda i: (0, i)),
            pl.BlockSpec((1, r), lambda i: (0, i)),
            pl.BlockSpec((1, 1), lambda i: (0, 0)),
        ],
        out_shape=[
            jax.ShapeDtypeStruct((E, CAP, S), jnp.float32),
            jax.ShapeDtypeStruct((E, CAP, S), jnp.int8),
            jax.ShapeDtypeStruct((1, S), jnp.int32),
            jax.ShapeDtypeStruct((1, S), jnp.int32),
            jax.ShapeDtypeStruct((1, S), jnp.float32),
            jax.ShapeDtypeStruct((1, 1), jnp.float32),
        ],
        scratch_shapes=[
            pltpu.VMEM((E, 1), jnp.float32),
            pltpu.VMEM((E, 1), jnp.float32),
        ],
    )(input, W)
    combine_t, dispatch_t, idx, loc, g1s, laux = out
    combine = jnp.transpose(combine_t, (2, 0, 1))
    dispatch = jnp.transpose(dispatch_t != 0, (2, 0, 1))
    return (laux[0, 0], combine, dispatch, idx.reshape(S), loc.reshape(S),
            g1s.reshape(S))
